# baseline (speedup 1.0000x reference)
import functools

import numpy as np
import jax
import jax.numpy as jnp
from jax import lax
from jax.experimental import pallas as pl
from jax.experimental.pallas import tpu as pltpu
from jax.experimental.pallas import tpu_sc as plsc

_MIN_DISABLED = 4
_MAX_DISABLED = 16


def _disabled_tofs(tof_count, min_c, max_c, seed=0):
    rng = np.random.RandomState(seed)
    disabled_count = int(rng.randint(min_c, max_c + 1))
    initial = int(rng.randint(0, tof_count))
    disabled = [initial]
    tof_list = rng.permutation(tof_count)
    tof_list = tof_list[tof_list != initial]
    for _ in range(disabled_count - 1):
        perm = rng.permutation(len(disabled))
        permuted = [disabled[i] for i in perm]
        opposite_found = False
        for cur in permuted:
            new_opp = (cur + tof_count // 2) % tof_count
            if new_opp not in disabled:
                disabled.append(int(new_opp))
                tof_list = tof_list[tof_list != new_opp]
                opposite_found = True
                break
        if not opposite_found:
            new_el = int(tof_list[0])
            tof_list = tof_list[tof_list != new_el]
            disabled.append(new_el)
    return np.asarray(disabled, dtype=np.int64)


_N_ROWS, _N_COLS = 65536, 512
_NC, _NS, _LANES = 2, 16, 16
_NW = _NC * _NS
_RPW = _N_ROWS // _NW
_C = 32
_NBUF = 4
_NCHUNK = _RPW // _C

_dis_np = _disabled_tofs(_N_COLS, _MIN_DISABLED, _MAX_DISABLED, 0)
_DIS16 = np.full((_LANES,), _dis_np[0], np.int32)
_DIS16[: len(_dis_np)] = _dis_np.astype(np.int32)


def _sc_body(img, dis, out, buf, dis_v, *sems):
    wid = lax.axis_index("s") * _NC + lax.axis_index("c")
    base = wid * _RPW
    pltpu.sync_copy(dis, dis_v)
    dvec = dis_v[...]
    zeros = jnp.zeros((_LANES,), jnp.float32)
    s_in = sems[:_NBUF]
    s_out = sems[_NBUF:]

    def start_in(b, chunk):
        pltpu.async_copy(img.at[pl.ds(base + chunk * _C, _C)], buf.at[b], s_in[b])

    def wait_in(b):
        pltpu.make_async_copy(img.at[pl.ds(0, _C)], buf.at[b], s_in[b]).wait()

    def start_out(b, chunk):
        pltpu.async_copy(buf.at[b], out.at[pl.ds(base + chunk * _C, _C)], s_out[b])

    def wait_out(b):
        pltpu.make_async_copy(buf.at[b], out.at[pl.ds(0, _C)], s_out[b]).wait()

    start_in(0, 0)
    start_in(1, 1)

    def group(g, carry):
        for b in range(_NBUF):
            i = g * _NBUF + b
            bp = (b + 2) % _NBUF

            @pl.when(jnp.logical_and(i >= 2, i + 2 < _NCHUNK))
            def _():
                wait_out(bp)
                start_in(bp, i + 2)

            @pl.when(jnp.logical_and(i < 2, i + 2 < _NCHUNK))
            def _():
                start_in(bp, i + 2)

            wait_in(b)
            for r in range(_C):
                plsc.store_scatter(
                    buf.at[b], [jnp.full((_LANES,), r, jnp.int32), dvec], zeros
                )
            start_out(b, i)
        return carry

    lax.fori_loop(0, _NCHUNK // _NBUF, group, 0)
    for b in range(_NBUF):
        wait_out(b)


@jax.jit
def _disable_tofs_sc(img, dis):
    mesh = plsc.VectorSubcoreMesh(core_axis_name="c", subcore_axis_name="s")
    return pl.kernel(
        _sc_body,
        out_type=jax.ShapeDtypeStruct((_N_ROWS, _N_COLS), jnp.float32),
        mesh=mesh,
        compiler_params=pltpu.CompilerParams(
            use_tc_tiling_on_sc=False, needs_layout_passes=False
        ),
        scratch_types=[
            pltpu.VMEM((_NBUF, _C, _N_COLS), jnp.float32),
            pltpu.VMEM((_LANES,), jnp.int32),
        ]
        + [pltpu.SemaphoreType.DMA] * (2 * _NBUF),
    )(img, dis)


def kernel(img):
    dis = jnp.asarray(_DIS16)
    return _disable_tofs_sc(img, dis)


# speedup vs baseline: 3.0255x; 3.0255x over previous
import functools

import numpy as np
import jax
import jax.numpy as jnp
from jax import lax
from jax.experimental import pallas as pl
from jax.experimental.pallas import tpu as pltpu
from jax.experimental.pallas import tpu_sc as plsc

_MIN_DISABLED = 4
_MAX_DISABLED = 16


def _disabled_tofs(tof_count, min_c, max_c, seed=0):
    rng = np.random.RandomState(seed)
    disabled_count = int(rng.randint(min_c, max_c + 1))
    initial = int(rng.randint(0, tof_count))
    disabled = [initial]
    tof_list = rng.permutation(tof_count)
    tof_list = tof_list[tof_list != initial]
    for _ in range(disabled_count - 1):
        perm = rng.permutation(len(disabled))
        permuted = [disabled[i] for i in perm]
        opposite_found = False
        for cur in permuted:
            new_opp = (cur + tof_count // 2) % tof_count
            if new_opp not in disabled:
                disabled.append(int(new_opp))
                tof_list = tof_list[tof_list != new_opp]
                opposite_found = True
                break
        if not opposite_found:
            new_el = int(tof_list[0])
            tof_list = tof_list[tof_list != new_el]
            disabled.append(new_el)
    return np.asarray(disabled, dtype=np.int64)


_N_ROWS, _N_COLS = 65536, 512
_NC, _NS, _LANES = 2, 16, 16
_NW = _NC * _NS
_RPW = _N_ROWS // _NW
_C = 32
_NBUF = 4
_NCHUNK = _RPW // _C

_dis_np = _disabled_tofs(_N_COLS, _MIN_DISABLED, _MAX_DISABLED, 0)
_DIS16 = np.full((_LANES,), _dis_np[0], np.int32)
_DIS16[: len(_dis_np)] = _dis_np.astype(np.int32)


def _sc_body(img, dis, out, buf, dis_v, *sems):
    wid = lax.axis_index("s") * _NC + lax.axis_index("c")
    base = wid * _RPW
    pltpu.sync_copy(dis, dis_v)
    dvec = dis_v[...]
    zeros = jnp.zeros((_LANES,), jnp.float32)
    s_in = sems[:_NBUF]
    s_out = sems[_NBUF:]

    def start_in(b, chunk):
        pltpu.async_copy(img.at[pl.ds(base + chunk * _C, _C)], buf.at[b], s_in[b])

    def wait_in(b):
        pltpu.make_async_copy(img.at[pl.ds(0, _C)], buf.at[b], s_in[b]).wait()

    def start_out(b, chunk):
        pltpu.async_copy(buf.at[b], out.at[pl.ds(base + chunk * _C, _C)], s_out[b])

    def wait_out(b):
        pltpu.make_async_copy(buf.at[b], out.at[pl.ds(0, _C)], s_out[b]).wait()

    start_in(0, 0)
    start_in(1, 1)

    def group(g, carry):
        for b in range(_NBUF):
            i = g * _NBUF + b
            bp = (b + 2) % _NBUF

            @pl.when(jnp.logical_and(i >= 2, i + 2 < _NCHUNK))
            def _():
                wait_out(bp)
                start_in(bp, i + 2)

            @pl.when(jnp.logical_and(i < 2, i + 2 < _NCHUNK))
            def _():
                start_in(bp, i + 2)

            wait_in(b)
            for r in range(_C):
                plsc.store_scatter(
                    buf.at[b], [jnp.full((_LANES,), r, jnp.int32), dvec], zeros
                )
            start_out(b, i)
        return carry

    lax.fori_loop(0, _NCHUNK // _NBUF, group, 0)
    for b in range(_NBUF):
        wait_out(b)


@jax.jit
def _disable_tofs_sc(img, dis):
    mesh = plsc.VectorSubcoreMesh(core_axis_name="c", subcore_axis_name="s")
    return pl.kernel(
        _sc_body,
        out_type=jax.ShapeDtypeStruct((_N_ROWS, _N_COLS), jnp.float32),
        mesh=mesh,
        compiler_params=pltpu.CompilerParams(needs_layout_passes=False),
        scratch_types=[
            pltpu.VMEM((_NBUF, _C, _N_COLS), jnp.float32),
            pltpu.VMEM((_LANES,), jnp.int32),
        ]
        + [pltpu.SemaphoreType.DMA] * (2 * _NBUF),
    )(img, dis)


def kernel(img):
    dis = jnp.asarray(_DIS16)
    return _disable_tofs_sc(img, dis)
